# baseline (speedup 1.0000x reference)
import functools
import jax
import jax.numpy as jnp
from jax import lax
from jax.experimental import pallas as pl
from jax.experimental.pallas import tpu as pltpu
from jax.experimental.pallas import tpu_sc as plsc

_K = 8192
_D = 256
_NW = 32
_ROWS = _K // _NW
_NCHUNK = 4
_CH = _ROWS // _NCHUNK


def _sc_copy(x_hbm, o_hbm, buf, in_sems, out_sems):
    wid = lax.axis_index("s") * 2 + lax.axis_index("c")
    base = wid * _ROWS
    ins = []
    outs = []
    for i in range(_NCHUNK):
        sl = pl.ds(base + i * _CH, _CH)
        ins.append(pltpu.make_async_copy(
            x_hbm.at[sl, :], buf.at[i], in_sems.at[i]))
        outs.append(pltpu.make_async_copy(
            buf.at[i], o_hbm.at[sl, :], out_sems.at[i]))
    for c in ins:
        c.start()
    for i in range(_NCHUNK):
        ins[i].wait()
        outs[i].start()
    for c in outs:
        c.wait()


def kernel(embedding_weight):
    mesh = plsc.VectorSubcoreMesh(core_axis_name="c", subcore_axis_name="s")
    run = functools.partial(
        pl.kernel,
        out_type=jax.ShapeDtypeStruct((_K, _D), jnp.float32),
        mesh=mesh,
        scratch_types=[
            pltpu.VMEM((_NCHUNK, _CH, _D), jnp.float32),
            pltpu.SemaphoreType.DMA((_NCHUNK,)),
            pltpu.SemaphoreType.DMA((_NCHUNK,)),
        ],
    )(_sc_copy)
    return run(embedding_weight)


# speedup vs baseline: 4.2127x; 4.2127x over previous
import jax
import jax.numpy as jnp
from jax.experimental import pallas as pl
from jax.experimental.pallas import tpu as pltpu

_NCHUNKS = 4


def _stream_copy(x_hbm, o_hbm, vmem, in_sems, out_sems):
    K = vmem.shape[0]
    rows = K // _NCHUNKS
    ins = []
    outs = []
    for i in range(_NCHUNKS):
        sl = pl.ds(i * rows, rows)
        ins.append(pltpu.make_async_copy(
            x_hbm.at[sl, :], vmem.at[sl, :], in_sems.at[i]))
        outs.append(pltpu.make_async_copy(
            vmem.at[sl, :], o_hbm.at[sl, :], out_sems.at[i]))
    for c in ins:
        c.start()
    for i in range(_NCHUNKS):
        ins[i].wait()
        outs[i].start()
    for c in outs:
        c.wait()


def kernel(embedding_weight):
    K, D = embedding_weight.shape
    return pl.pallas_call(
        _stream_copy,
        in_specs=[pl.BlockSpec(memory_space=pl.ANY)],
        out_specs=pl.BlockSpec(memory_space=pl.ANY),
        out_shape=jax.ShapeDtypeStruct((K, D), embedding_weight.dtype),
        scratch_shapes=[
            pltpu.VMEM((K, D), embedding_weight.dtype),
            pltpu.SemaphoreType.DMA((_NCHUNKS,)),
            pltpu.SemaphoreType.DMA((_NCHUNKS,)),
        ],
    )(embedding_weight)
